# baseline (speedup 1.0000x reference)
import functools

import jax
import jax.numpy as jnp
from jax import lax
from jax.experimental import pallas as pl
from jax.experimental.pallas import tpu as pltpu
from jax.experimental.pallas import tpu_sc as plsc

MASK_RATE = 0.5
VIOLATION_WEIGHT = 0.3
RANDOM_WEIGHT = 0.7
N = 100000
D = 128
NUM_MASK = 50000

NT = 16
NL = 16
NV = 391
CH = NL * NV
NPAD = NT * CH
NTRASH = NPAD + NL
NCH = 52
KEEP_DUMMY = 50048
KEEP_BUF = KEEP_DUMMY + NT * NL

OUT_CHUNK = 3128
LAST_CHUNK = NUM_MASK - 15 * OUT_CHUNK


def _sc_body(keys_hbm, mask_hbm, keep_hbm,
             v_keys, v_vals, v_hist, v_offs, v_grid, v_t256, v_pre, v_base,
             v_pos, v_kbuf, v_vbuf, v_fbuf,
             sp_keys_a, sp_vals_a, sp_keys_b, sp_vals_b, sp_T, sp_ind,
             sp_keep, sp_cnt, sem_a, sem_b, sem_c, sem_d):
    _SEMS = (sem_a, sem_b, sem_c, sem_d)
    tid = lax.axis_index("s")
    iota = lax.iota(jnp.int32, NL)
    lane_sub = iota * NV
    lane_hist = iota * 256
    zeros16 = jnp.zeros((NL,), jnp.int32)
    ones16 = jnp.ones((NL,), jnp.int32)

    def _z(i, _):
        v_vals[pl.ds(i * NL, NL)] = zeros16
        return 0
    lax.fori_loop(0, NV, _z, 0, unroll=8)
    pltpu.sync_copy(v_vals.at[pl.ds(0, CH)], sp_ind.at[pl.ds(tid * CH, CH)])

    sp_bufs = [(sp_keys_a, sp_vals_a), (sp_keys_b, sp_vals_b)]

    for p in range(4):
        shift = 8 * p
        if p == 0:
            keys_src = None
        else:
            keys_src, vals_src = sp_bufs[(p + 1) % 2]
        keys_dst, vals_dst = sp_bufs[p % 2]

        if p == 0:
            pltpu.sync_copy(keys_hbm.at[pl.ds(tid * CH, CH)], v_keys)
        else:
            pltpu.sync_copy(keys_src.at[pl.ds(tid * CH, CH)], v_keys)
            pltpu.sync_copy(vals_src.at[pl.ds(tid * CH, CH)], v_vals)

        def _zh(i, _):
            v_hist[pl.ds(i * NL, NL)] = zeros16
            return 0
        lax.fori_loop(0, 256, _zh, 0, unroll=8)

        def _hist(v, _):
            k = plsc.load_gather(v_keys, [lane_sub + v])
            ku = plsc.bitcast(k, jnp.uint32)
            d = plsc.bitcast((ku >> shift) & 0xFF, jnp.int32)
            plsc.addupdate_scatter(v_hist, [lane_hist + d], ones16)
            return 0
        lax.fori_loop(0, NV, _hist, 0, unroll=4)

        for dv in range(16):
            acc = v_hist[pl.ds(dv * NL, NL)]
            for l in range(1, NL):
                acc = acc + v_hist[pl.ds(l * 256 + dv * NL, NL)]
            v_t256[pl.ds(dv * NL, NL)] = acc
        pltpu.sync_copy(v_t256.at[pl.ds(0, 256)], sp_T.at[pl.ds(tid * 256, 256)])
        plsc.subcore_barrier()

        pltpu.sync_copy(sp_T.at[pl.ds(0, 4096)], v_grid)
        for dv in range(16):
            tot = zeros16
            pre = zeros16
            for t in range(NT):
                x = v_grid[pl.ds(t * 256 + dv * NL, NL)]
                tot = tot + x
                pre = pre + x * jnp.where(t < tid, 1, 0).astype(jnp.int32)
            v_t256[pl.ds(dv * NL, NL)] = tot
            v_pre[pl.ds(dv * NL, NL)] = pre
        carry = jnp.int32(0)
        for dv in range(16):
            g = v_t256[pl.ds(dv * NL, NL)]
            excl = plsc.cumsum(g) - g
            v_base[pl.ds(dv * NL, NL)] = excl + carry
            carry = carry + jnp.sum(g)
        for dv in range(16):
            run = v_base[pl.ds(dv * NL, NL)] + v_pre[pl.ds(dv * NL, NL)]
            for l in range(NL):
                v_offs[pl.ds(l * 256 + dv * NL, NL)] = run
                run = run + v_hist[pl.ds(l * 256 + dv * NL, NL)]

        last = p == 3
        pad_idx = NPAD + iota

        def _issue(par, sem):
            if not last:
                pltpu.async_copy(v_kbuf.at[par], keys_dst.at[v_pos.at[par]], sem)
            pltpu.async_copy(v_vbuf.at[par], vals_dst.at[v_pos.at[par]], sem)
            if last:
                pltpu.async_copy(v_fbuf.at[par], sp_ind.at[v_vbuf.at[par]], sem)

        def _drain(par, sem):
            if not last:
                pltpu.make_async_copy(v_kbuf.at[par], keys_dst.at[v_pos.at[par]], sem).wait()
            pltpu.make_async_copy(v_vbuf.at[par], vals_dst.at[v_pos.at[par]], sem).wait()
            if last:
                pltpu.make_async_copy(v_fbuf.at[par], sp_ind.at[v_vbuf.at[par]], sem).wait()

        def _chunk(c2, _):
            for par in range(4):
                sem = _SEMS[par]

                @pl.when(c2 > 0)
                def _w():
                    _drain(par, sem)

                for j in range(8):
                    v = (c2 * 4 + par) * 8 + j
                    valid = v < NV
                    vv = jnp.where(valid, v, 0)
                    sl = lane_sub + vv
                    k = plsc.load_gather(v_keys, [sl])
                    if p == 0:
                        val = tid * CH + sl
                    else:
                        val = plsc.load_gather(v_vals, [sl])
                    ku = plsc.bitcast(k, jnp.uint32)
                    d = plsc.bitcast((ku >> shift) & 0xFF, jnp.int32)
                    addr = lane_hist + d
                    pos = plsc.load_gather(v_offs, [addr])
                    vmul = jnp.where(valid, 1, 0).astype(jnp.int32)
                    plsc.store_scatter(v_offs, [addr], pos + vmul)
                    pos = jnp.where(valid, pos, pad_idx)
                    val = jnp.where(valid, val, pad_idx)
                    v_pos[par, pl.ds(j * NL, NL)] = pos
                    v_kbuf[par, pl.ds(j * NL, NL)] = k
                    v_vbuf[par, pl.ds(j * NL, NL)] = val
                    if last:
                        v_fbuf[par, pl.ds(j * NL, NL)] = jnp.where(
                            pos < NUM_MASK, 1, 0).astype(jnp.int32) * vmul
                _issue(par, sem)
            return 0
        lax.fori_loop(0, NCH // 4, _chunk, 0)
        for par in range(4):
            _drain(par, _SEMS[par])
        plsc.subcore_barrier()

    pltpu.sync_copy(sp_ind.at[pl.ds(tid * CH, CH)], v_keys)
    gbase = tid * CH

    def _cnt(i, c):
        ind = v_keys[pl.ds(i * NL, NL)]
        g = gbase + i * NL + iota
        f = jnp.where((ind == 0) & (g < N), 1, 0).astype(jnp.int32)
        return c + jnp.sum(f)
    cnt = lax.fori_loop(0, NV, _cnt, jnp.int32(0), unroll=4)
    v_t256[pl.ds(0, NL)] = jnp.where(iota == 0, cnt, 0).astype(jnp.int32)
    pltpu.sync_copy(v_t256.at[pl.ds(0, NL)], sp_cnt.at[pl.ds(tid * NL, NL)])
    plsc.subcore_barrier()
    pltpu.sync_copy(sp_cnt.at[pl.ds(0, 256)], v_t256.at[pl.ds(0, 256)])
    off = jnp.int32(0)
    for t in range(NT):
        x = v_t256[pl.ds(t * NL, NL)]
        off = off + jnp.sum(x) * jnp.where(t < tid, 1, 0).astype(jnp.int32)

    def _keep(c2, run):
        for par in range(2):
            sem = sem_a if par == 0 else sem_b

            @pl.when(c2 > 0)
            def _wk():
                pltpu.make_async_copy(
                    v_vbuf.at[par], sp_keep.at[v_pos.at[par]], sem).wait()

            for j in range(8):
                i = (c2 * 2 + par) * 8 + j
                valid = i < NV
                ii = jnp.where(valid, i, 0)
                ind = plsc.load_gather(v_keys, [ii * NL + iota])
                g = gbase + ii * NL + iota
                f = jnp.where((ind == 0) & (g < N) & valid, 1, 0).astype(jnp.int32)
                excl = plsc.cumsum(f) - f
                pos = run + excl
                pos = jnp.where(f == 1, pos, KEEP_DUMMY + tid * NL + iota)
                run = run + jnp.sum(f)
                v_pos[par, pl.ds(j * NL, NL)] = pos
                v_vbuf[par, pl.ds(j * NL, NL)] = g
            pltpu.async_copy(v_vbuf.at[par], sp_keep.at[v_pos.at[par]], sem)
        return run
    lax.fori_loop(0, NCH // 2, _keep, off)
    pltpu.make_async_copy(v_vbuf.at[0], sp_keep.at[v_pos.at[0]], sem_a).wait()
    pltpu.make_async_copy(v_vbuf.at[1], sp_keep.at[v_pos.at[1]], sem_b).wait()
    plsc.subcore_barrier()

    _, sp_vals_fin = sp_bufs[1]

    @pl.when(tid < 15)
    def _copy_main():
        o = pl.multiple_of(tid * OUT_CHUNK, 8)
        pltpu.sync_copy(sp_vals_fin.at[pl.ds(o, OUT_CHUNK)], v_vals.at[pl.ds(0, OUT_CHUNK)])
        pltpu.sync_copy(v_vals.at[pl.ds(0, OUT_CHUNK)], mask_hbm.at[pl.ds(o, OUT_CHUNK)])
        pltpu.sync_copy(sp_keep.at[pl.ds(o, OUT_CHUNK)], v_keys.at[pl.ds(0, OUT_CHUNK)])
        pltpu.sync_copy(v_keys.at[pl.ds(0, OUT_CHUNK)], keep_hbm.at[pl.ds(o, OUT_CHUNK)])

    @pl.when(tid == 15)
    def _copy_last():
        o = 15 * OUT_CHUNK
        pltpu.sync_copy(sp_vals_fin.at[pl.ds(o, LAST_CHUNK)], v_vals.at[pl.ds(0, LAST_CHUNK)])
        pltpu.sync_copy(v_vals.at[pl.ds(0, LAST_CHUNK)], mask_hbm.at[pl.ds(o, LAST_CHUNK)])
        pltpu.sync_copy(sp_keep.at[pl.ds(o, LAST_CHUNK)], v_keys.at[pl.ds(0, LAST_CHUNK)])
        pltpu.sync_copy(v_keys.at[pl.ds(0, LAST_CHUNK)], keep_hbm.at[pl.ds(o, LAST_CHUNK)])


@functools.partial(jax.jit, static_argnums=())
def _sc_sort(keys_pad):
    mesh = plsc.VectorSubcoreMesh(core_axis_name="c", subcore_axis_name="s",
                                  num_cores=1)
    f = pl.kernel(
        _sc_body,
        compiler_params=pltpu.CompilerParams(needs_layout_passes=False),
        cost_estimate=pl.CostEstimate(
            flops=4_000_000, bytes_accessed=16_000_000, transcendentals=0),
        out_type=(
            jax.ShapeDtypeStruct((NUM_MASK,), jnp.int32),
            jax.ShapeDtypeStruct((NUM_MASK,), jnp.int32),
        ),
        mesh=mesh,
        scratch_types=dict(
            v_keys=pltpu.VMEM((CH,), jnp.int32),
            v_vals=pltpu.VMEM((CH,), jnp.int32),
            v_hist=pltpu.VMEM((4096,), jnp.int32),
            v_offs=pltpu.VMEM((4096,), jnp.int32),
            v_grid=pltpu.VMEM((4096,), jnp.int32),
            v_t256=pltpu.VMEM((256,), jnp.int32),
            v_pre=pltpu.VMEM((256,), jnp.int32),
            v_base=pltpu.VMEM((256,), jnp.int32),
            v_pos=pltpu.VMEM((4, 128), jnp.int32),
            v_kbuf=pltpu.VMEM((4, 128), jnp.int32),
            v_vbuf=pltpu.VMEM((4, 128), jnp.int32),
            v_fbuf=pltpu.VMEM((4, 128), jnp.int32),
            sp_keys_a=pltpu.VMEM_SHARED((NTRASH,), jnp.int32),
            sp_vals_a=pltpu.VMEM_SHARED((NTRASH,), jnp.int32),
            sp_keys_b=pltpu.VMEM_SHARED((NTRASH,), jnp.int32),
            sp_vals_b=pltpu.VMEM_SHARED((NTRASH,), jnp.int32),
            sp_T=pltpu.VMEM_SHARED((4096,), jnp.int32),
            sp_ind=pltpu.VMEM_SHARED((NTRASH,), jnp.int32),
            sp_keep=pltpu.VMEM_SHARED((KEEP_BUF,), jnp.int32),
            sp_cnt=pltpu.VMEM_SHARED((256,), jnp.int32),
            sem_a=pltpu.SemaphoreType.DMA,
            sem_b=pltpu.SemaphoreType.DMA,
            sem_c=pltpu.SemaphoreType.DMA,
            sem_d=pltpu.SemaphoreType.DMA,
        ),
    )
    return f(keys_pad)


def _tc_copy_body(f_ref, o_ref):
    o_ref[...] = f_ref[...]


def _tc_copy(features):
    blk = 2000
    grid = (N // blk,)
    return pl.pallas_call(
        _tc_copy_body,
        grid=grid,
        in_specs=[pl.BlockSpec((blk, D), lambda i: (i, 0))],
        out_specs=pl.BlockSpec((blk, D), lambda i: (i, 0)),
        out_shape=jax.ShapeDtypeStruct((N, D), jnp.float32),
    )(features)


NW = 32
NFULL_CHUNK = NUM_MASK // 128
REM = NUM_MASK - NFULL_CHUNK * 128
REM_W = NFULL_CHUNK % NW
SCAT_IT = (NFULL_CHUNK + NW - 1) // NW


def _scatter_body(mask_hbm, feat_in_hbm, tok_hbm, out_hbm, v_idx, v_rem,
                  v_tok, sem, sem_r):
    del feat_in_hbm
    wid = lax.axis_index("s") * 2 + lax.axis_index("c")
    pltpu.sync_copy(tok_hbm, v_tok)

    def _go(i, _):
        c = wid + i * NW

        @pl.when(c < NFULL_CHUNK)
        def _full():
            pltpu.sync_copy(mask_hbm.at[pl.ds(c * 128, 128)], v_idx.at[i])
            pltpu.async_copy(v_tok, out_hbm.at[v_idx.at[i]], sem)
        return 0
    lax.fori_loop(0, SCAT_IT, _go, 0)

    @pl.when(wid == REM_W)
    def _rem():
        o = pl.multiple_of(NFULL_CHUNK * 128, 8)
        pltpu.sync_copy(mask_hbm.at[pl.ds(o, REM)], v_rem)
        pltpu.async_copy(v_tok.at[pl.ds(0, REM)], out_hbm.at[v_rem], sem_r).wait()

    def _drain(i, _):
        c = wid + i * NW

        @pl.when(c < NFULL_CHUNK)
        def _w():
            pltpu.make_async_copy(v_tok, out_hbm.at[v_idx.at[i]], sem).wait()
        return 0
    lax.fori_loop(0, SCAT_IT, _drain, 0)


def _sc_scatter(mask_nodes, feat_copy, tok128):
    from jax._src.pallas import mpmd as _mpmd
    mesh = plsc.VectorSubcoreMesh(core_axis_name="c", subcore_axis_name="s",
                                  num_cores=2)
    f = _mpmd._mpmd_map(
        [(mesh, _scatter_body)],
        (jax.ShapeDtypeStruct((N, D), jnp.float32),),
        input_output_aliases={1: 0},
        compiler_params=pltpu.CompilerParams(needs_layout_passes=False),
        scratch_types=dict(
            v_idx=pltpu.VMEM((SCAT_IT, 128), jnp.int32),
            v_rem=pltpu.VMEM((REM,), jnp.int32),
            v_tok=pltpu.VMEM((128, D), jnp.float32),
            sem=pltpu.SemaphoreType.DMA,
            sem_r=pltpu.SemaphoreType.DMA,
        ),
    )
    (out,) = f(mask_nodes, feat_copy, tok128)
    return out


@functools.lru_cache(maxsize=1)
def _gumbel_const():
    skey = jax.random.key(42)
    return jax.random.gumbel(skey, (N,), dtype=jnp.float32)


def kernel(features, cic_scores, mask_token):
    weights = jnp.array([0.25, 0.25, 0.25, 0.25], dtype=jnp.float32)
    weighted = 1.0 - weights[None, :] * jnp.clip(cic_scores, 0.0, 1.0)
    total_scores = 1.0 - jnp.prod(weighted, axis=1)
    total_scores = jnp.clip(total_scores.astype(jnp.float32), 0.0, 1.0)
    violation_probs = total_scores + 1e-06
    random_probs = jnp.ones(N, dtype=jnp.float32)
    probs = VIOLATION_WEIGHT * violation_probs + RANDOM_WEIGHT * random_probs
    probs = probs / jnp.sum(probs)
    scores = jnp.log(probs) + _gumbel_const()

    b = lax.bitcast_convert_type(scores, jnp.int32)
    kp = jnp.where(b >= 0, ~b & 0x7FFFFFFF, b).astype(jnp.int32)
    keys_pad = jnp.concatenate(
        [kp, jnp.full((NPAD - N,), -1, jnp.int32)])

    feat_copy = _tc_copy(features)
    mask_nodes, keep_nodes = _sc_sort(keys_pad)
    tok128 = jnp.broadcast_to(mask_token, (128, D))
    new_features = _sc_scatter(mask_nodes, feat_copy, tok128)
    return (new_features, mask_nodes, keep_nodes)


# speedup vs baseline: 1.0431x; 1.0431x over previous
import functools

import jax
import jax.numpy as jnp
from jax import lax
from jax.experimental import pallas as pl
from jax.experimental.pallas import tpu as pltpu
from jax.experimental.pallas import tpu_sc as plsc

MASK_RATE = 0.5
VIOLATION_WEIGHT = 0.3
RANDOM_WEIGHT = 0.7
N = 100000
D = 128
NUM_MASK = 50000

NT = 16
NL = 16
NV = 391
CH = NL * NV
NPAD = NT * CH
NTRASH = NPAD + NL
NCH = 50
KEEP_DUMMY = 50048
KEEP_BUF = KEEP_DUMMY + NT * NL

OUT_CHUNK = 3128
LAST_CHUNK = NUM_MASK - 15 * OUT_CHUNK


def _sc_body(keys_hbm, mask_hbm, keep_hbm,
             v_keys, v_vals, v_hist, v_offs, v_grid, v_t256, v_pre, v_base,
             v_pos, v_kbuf, v_vbuf, v_fbuf,
             sp_keys_a, sp_vals_a, sp_keys_b, sp_vals_b, sp_T, sp_ind,
             sp_keep, sp_cnt, sem_a, sem_b, sem_c, sem_d):
    _SEMS = (sem_a, sem_b, sem_c, sem_d)
    tid = lax.axis_index("s")
    iota = lax.iota(jnp.int32, NL)
    lane_sub = iota * NV
    lane_hist = iota * 256
    zeros16 = jnp.zeros((NL,), jnp.int32)
    ones16 = jnp.ones((NL,), jnp.int32)

    def _z(i, _):
        v_vals[pl.ds(i * NL, NL)] = zeros16
        return 0
    lax.fori_loop(0, NV, _z, 0, unroll=8)
    pltpu.sync_copy(v_vals.at[pl.ds(0, CH)], sp_ind.at[pl.ds(tid * CH, CH)])

    sp_bufs = [(sp_keys_a, sp_vals_a), (sp_keys_b, sp_vals_b)]

    for p in range(4):
        shift = 8 * p
        if p == 0:
            keys_src = None
        else:
            keys_src, vals_src = sp_bufs[(p + 1) % 2]
        keys_dst, vals_dst = sp_bufs[p % 2]

        if p == 0:
            pltpu.sync_copy(keys_hbm.at[pl.ds(tid * CH, CH)], v_keys)
        else:
            pltpu.sync_copy(keys_src.at[pl.ds(tid * CH, CH)], v_keys)
            pltpu.sync_copy(vals_src.at[pl.ds(tid * CH, CH)], v_vals)

        def _zh(i, _):
            v_hist[pl.ds(i * NL, NL)] = zeros16
            return 0
        lax.fori_loop(0, 256, _zh, 0, unroll=8)

        def _hist(v, _):
            k = plsc.load_gather(v_keys, [lane_sub + v])
            ku = plsc.bitcast(k, jnp.uint32)
            d = plsc.bitcast((ku >> shift) & 0xFF, jnp.int32)
            plsc.addupdate_scatter(v_hist, [lane_hist + d], ones16)
            return 0
        lax.fori_loop(0, NV, _hist, 0, unroll=4)

        for dv in range(16):
            acc = v_hist[pl.ds(dv * NL, NL)]
            for l in range(1, NL):
                acc = acc + v_hist[pl.ds(l * 256 + dv * NL, NL)]
            v_t256[pl.ds(dv * NL, NL)] = acc
        pltpu.sync_copy(v_t256.at[pl.ds(0, 256)], sp_T.at[pl.ds(tid * 256, 256)])
        plsc.subcore_barrier()

        pltpu.sync_copy(sp_T.at[pl.ds(0, 4096)], v_grid)
        for dv in range(16):
            tot = zeros16
            pre = zeros16
            for t in range(NT):
                x = v_grid[pl.ds(t * 256 + dv * NL, NL)]
                tot = tot + x
                pre = pre + x * jnp.where(t < tid, 1, 0).astype(jnp.int32)
            v_t256[pl.ds(dv * NL, NL)] = tot
            v_pre[pl.ds(dv * NL, NL)] = pre
        carry = jnp.int32(0)
        for dv in range(16):
            g = v_t256[pl.ds(dv * NL, NL)]
            excl = plsc.cumsum(g) - g
            v_base[pl.ds(dv * NL, NL)] = excl + carry
            carry = carry + jnp.sum(g)
        for dv in range(16):
            run = v_base[pl.ds(dv * NL, NL)] + v_pre[pl.ds(dv * NL, NL)]
            for l in range(NL):
                v_offs[pl.ds(l * 256 + dv * NL, NL)] = run
                run = run + v_hist[pl.ds(l * 256 + dv * NL, NL)]

        last = p == 3
        pad_idx = NPAD + iota

        def _issue(par, sem):
            if not last:
                pltpu.async_copy(v_kbuf.at[par], keys_dst.at[v_pos.at[par]], sem)
            pltpu.async_copy(v_vbuf.at[par], vals_dst.at[v_pos.at[par]], sem)
            if last:
                pltpu.async_copy(v_fbuf.at[par], sp_ind.at[v_vbuf.at[par]], sem)

        def _drain(par, sem):
            if not last:
                pltpu.make_async_copy(v_kbuf.at[par], keys_dst.at[v_pos.at[par]], sem).wait()
            pltpu.make_async_copy(v_vbuf.at[par], vals_dst.at[v_pos.at[par]], sem).wait()
            if last:
                pltpu.make_async_copy(v_fbuf.at[par], sp_ind.at[v_vbuf.at[par]], sem).wait()

        def _chunk(c2, _):
            for par in range(2):
                sem = _SEMS[par]

                @pl.when(c2 > 0)
                def _w():
                    _drain(par, sem)

                for j in range(8):
                    v = (c2 * 2 + par) * 8 + j
                    valid = v < NV
                    vv = jnp.where(valid, v, 0)
                    sl = lane_sub + vv
                    k = plsc.load_gather(v_keys, [sl])
                    if p == 0:
                        val = tid * CH + sl
                    else:
                        val = plsc.load_gather(v_vals, [sl])
                    ku = plsc.bitcast(k, jnp.uint32)
                    d = plsc.bitcast((ku >> shift) & 0xFF, jnp.int32)
                    addr = lane_hist + d
                    pos = plsc.load_gather(v_offs, [addr])
                    vmul = jnp.where(valid, 1, 0).astype(jnp.int32)
                    plsc.store_scatter(v_offs, [addr], pos + vmul)
                    pos = jnp.where(valid, pos, pad_idx)
                    val = jnp.where(valid, val, pad_idx)
                    v_pos[par, pl.ds(j * NL, NL)] = pos
                    v_kbuf[par, pl.ds(j * NL, NL)] = k
                    v_vbuf[par, pl.ds(j * NL, NL)] = val
                    if last:
                        v_fbuf[par, pl.ds(j * NL, NL)] = jnp.where(
                            pos < NUM_MASK, 1, 0).astype(jnp.int32) * vmul
                _issue(par, sem)
            return 0
        lax.fori_loop(0, NCH // 2, _chunk, 0)
        for par in range(2):
            _drain(par, _SEMS[par])
        plsc.subcore_barrier()

    pltpu.sync_copy(sp_ind.at[pl.ds(tid * CH, CH)], v_keys)
    gbase = tid * CH

    def _cnt(i, c):
        ind = v_keys[pl.ds(i * NL, NL)]
        g = gbase + i * NL + iota
        f = jnp.where((ind == 0) & (g < N), 1, 0).astype(jnp.int32)
        return c + jnp.sum(f)
    cnt = lax.fori_loop(0, NV, _cnt, jnp.int32(0), unroll=4)
    v_t256[pl.ds(0, NL)] = jnp.where(iota == 0, cnt, 0).astype(jnp.int32)
    pltpu.sync_copy(v_t256.at[pl.ds(0, NL)], sp_cnt.at[pl.ds(tid * NL, NL)])
    plsc.subcore_barrier()
    pltpu.sync_copy(sp_cnt.at[pl.ds(0, 256)], v_t256.at[pl.ds(0, 256)])
    off = jnp.int32(0)
    for t in range(NT):
        x = v_t256[pl.ds(t * NL, NL)]
        off = off + jnp.sum(x) * jnp.where(t < tid, 1, 0).astype(jnp.int32)

    def _keep(c2, run):
        for par in range(2):
            sem = sem_a if par == 0 else sem_b

            @pl.when(c2 > 0)
            def _wk():
                pltpu.make_async_copy(
                    v_vbuf.at[par], sp_keep.at[v_pos.at[par]], sem).wait()

            for j in range(8):
                i = (c2 * 2 + par) * 8 + j
                valid = i < NV
                ii = jnp.where(valid, i, 0)
                ind = plsc.load_gather(v_keys, [ii * NL + iota])
                g = gbase + ii * NL + iota
                f = jnp.where((ind == 0) & (g < N) & valid, 1, 0).astype(jnp.int32)
                excl = plsc.cumsum(f) - f
                pos = run + excl
                pos = jnp.where(f == 1, pos, KEEP_DUMMY + tid * NL + iota)
                run = run + jnp.sum(f)
                v_pos[par, pl.ds(j * NL, NL)] = pos
                v_vbuf[par, pl.ds(j * NL, NL)] = g
            pltpu.async_copy(v_vbuf.at[par], sp_keep.at[v_pos.at[par]], sem)
        return run
    lax.fori_loop(0, NCH // 2, _keep, off)
    pltpu.make_async_copy(v_vbuf.at[0], sp_keep.at[v_pos.at[0]], sem_a).wait()
    pltpu.make_async_copy(v_vbuf.at[1], sp_keep.at[v_pos.at[1]], sem_b).wait()
    plsc.subcore_barrier()

    _, sp_vals_fin = sp_bufs[1]

    @pl.when(tid < 15)
    def _copy_main():
        o = pl.multiple_of(tid * OUT_CHUNK, 8)
        pltpu.sync_copy(sp_vals_fin.at[pl.ds(o, OUT_CHUNK)], v_vals.at[pl.ds(0, OUT_CHUNK)])
        pltpu.sync_copy(v_vals.at[pl.ds(0, OUT_CHUNK)], mask_hbm.at[pl.ds(o, OUT_CHUNK)])
        pltpu.sync_copy(sp_keep.at[pl.ds(o, OUT_CHUNK)], v_keys.at[pl.ds(0, OUT_CHUNK)])
        pltpu.sync_copy(v_keys.at[pl.ds(0, OUT_CHUNK)], keep_hbm.at[pl.ds(o, OUT_CHUNK)])

    @pl.when(tid == 15)
    def _copy_last():
        o = 15 * OUT_CHUNK
        pltpu.sync_copy(sp_vals_fin.at[pl.ds(o, LAST_CHUNK)], v_vals.at[pl.ds(0, LAST_CHUNK)])
        pltpu.sync_copy(v_vals.at[pl.ds(0, LAST_CHUNK)], mask_hbm.at[pl.ds(o, LAST_CHUNK)])
        pltpu.sync_copy(sp_keep.at[pl.ds(o, LAST_CHUNK)], v_keys.at[pl.ds(0, LAST_CHUNK)])
        pltpu.sync_copy(v_keys.at[pl.ds(0, LAST_CHUNK)], keep_hbm.at[pl.ds(o, LAST_CHUNK)])


@functools.partial(jax.jit, static_argnums=())
def _sc_sort(keys_pad):
    mesh = plsc.VectorSubcoreMesh(core_axis_name="c", subcore_axis_name="s",
                                  num_cores=1)
    f = pl.kernel(
        _sc_body,
        compiler_params=pltpu.CompilerParams(needs_layout_passes=False),
        cost_estimate=pl.CostEstimate(
            flops=4_000_000, bytes_accessed=16_000_000, transcendentals=0),
        out_type=(
            jax.ShapeDtypeStruct((NUM_MASK,), jnp.int32),
            jax.ShapeDtypeStruct((NUM_MASK,), jnp.int32),
        ),
        mesh=mesh,
        scratch_types=dict(
            v_keys=pltpu.VMEM((CH,), jnp.int32),
            v_vals=pltpu.VMEM((CH,), jnp.int32),
            v_hist=pltpu.VMEM((4096,), jnp.int32),
            v_offs=pltpu.VMEM((4096,), jnp.int32),
            v_grid=pltpu.VMEM((4096,), jnp.int32),
            v_t256=pltpu.VMEM((256,), jnp.int32),
            v_pre=pltpu.VMEM((256,), jnp.int32),
            v_base=pltpu.VMEM((256,), jnp.int32),
            v_pos=pltpu.VMEM((4, 128), jnp.int32),
            v_kbuf=pltpu.VMEM((4, 128), jnp.int32),
            v_vbuf=pltpu.VMEM((4, 128), jnp.int32),
            v_fbuf=pltpu.VMEM((4, 128), jnp.int32),
            sp_keys_a=pltpu.VMEM_SHARED((NTRASH,), jnp.int32),
            sp_vals_a=pltpu.VMEM_SHARED((NTRASH,), jnp.int32),
            sp_keys_b=pltpu.VMEM_SHARED((NTRASH,), jnp.int32),
            sp_vals_b=pltpu.VMEM_SHARED((NTRASH,), jnp.int32),
            sp_T=pltpu.VMEM_SHARED((4096,), jnp.int32),
            sp_ind=pltpu.VMEM_SHARED((NTRASH,), jnp.int32),
            sp_keep=pltpu.VMEM_SHARED((KEEP_BUF,), jnp.int32),
            sp_cnt=pltpu.VMEM_SHARED((256,), jnp.int32),
            sem_a=pltpu.SemaphoreType.DMA,
            sem_b=pltpu.SemaphoreType.DMA,
            sem_c=pltpu.SemaphoreType.DMA,
            sem_d=pltpu.SemaphoreType.DMA,
        ),
    )
    return f(keys_pad)


def _tc_copy_body(f_ref, o_ref):
    o_ref[...] = f_ref[...]


def _tc_copy(features):
    blk = 2000
    grid = (N // blk,)
    return pl.pallas_call(
        _tc_copy_body,
        grid=grid,
        in_specs=[pl.BlockSpec((blk, D), lambda i: (i, 0))],
        out_specs=pl.BlockSpec((blk, D), lambda i: (i, 0)),
        out_shape=jax.ShapeDtypeStruct((N, D), jnp.float32),
    )(features)


NW = 32
NFULL_CHUNK = NUM_MASK // 128
REM = NUM_MASK - NFULL_CHUNK * 128
REM_W = NFULL_CHUNK % NW
SCAT_IT = (NFULL_CHUNK + NW - 1) // NW


def _scatter_body(mask_hbm, feat_in_hbm, tok_hbm, out_hbm, v_idx, v_rem,
                  v_tok, sem, sem_r):
    del feat_in_hbm
    wid = lax.axis_index("s") * 2 + lax.axis_index("c")
    pltpu.sync_copy(tok_hbm, v_tok)

    def _go(i, _):
        c = wid + i * NW

        @pl.when(c < NFULL_CHUNK)
        def _full():
            pltpu.sync_copy(mask_hbm.at[pl.ds(c * 128, 128)], v_idx.at[i])
            pltpu.async_copy(v_tok, out_hbm.at[v_idx.at[i]], sem)
        return 0
    lax.fori_loop(0, SCAT_IT, _go, 0)

    @pl.when(wid == REM_W)
    def _rem():
        o = pl.multiple_of(NFULL_CHUNK * 128, 8)
        pltpu.sync_copy(mask_hbm.at[pl.ds(o, REM)], v_rem)
        pltpu.async_copy(v_tok.at[pl.ds(0, REM)], out_hbm.at[v_rem], sem_r).wait()

    def _drain(i, _):
        c = wid + i * NW

        @pl.when(c < NFULL_CHUNK)
        def _w():
            pltpu.make_async_copy(v_tok, out_hbm.at[v_idx.at[i]], sem).wait()
        return 0
    lax.fori_loop(0, SCAT_IT, _drain, 0)


def _sc_scatter(mask_nodes, feat_copy, tok128):
    from jax._src.pallas import mpmd as _mpmd
    mesh = plsc.VectorSubcoreMesh(core_axis_name="c", subcore_axis_name="s",
                                  num_cores=2)
    f = _mpmd._mpmd_map(
        [(mesh, _scatter_body)],
        (jax.ShapeDtypeStruct((N, D), jnp.float32),),
        input_output_aliases={1: 0},
        compiler_params=pltpu.CompilerParams(needs_layout_passes=False),
        scratch_types=dict(
            v_idx=pltpu.VMEM((SCAT_IT, 128), jnp.int32),
            v_rem=pltpu.VMEM((REM,), jnp.int32),
            v_tok=pltpu.VMEM((128, D), jnp.float32),
            sem=pltpu.SemaphoreType.DMA,
            sem_r=pltpu.SemaphoreType.DMA,
        ),
    )
    (out,) = f(mask_nodes, feat_copy, tok128)
    return out


@functools.lru_cache(maxsize=1)
def _gumbel_const():
    skey = jax.random.key(42)
    return jax.random.gumbel(skey, (N,), dtype=jnp.float32)


def kernel(features, cic_scores, mask_token):
    weights = jnp.array([0.25, 0.25, 0.25, 0.25], dtype=jnp.float32)
    weighted = 1.0 - weights[None, :] * jnp.clip(cic_scores, 0.0, 1.0)
    total_scores = 1.0 - jnp.prod(weighted, axis=1)
    total_scores = jnp.clip(total_scores.astype(jnp.float32), 0.0, 1.0)
    violation_probs = total_scores + 1e-06
    random_probs = jnp.ones(N, dtype=jnp.float32)
    probs = VIOLATION_WEIGHT * violation_probs + RANDOM_WEIGHT * random_probs
    probs = probs / jnp.sum(probs)
    scores = jnp.log(probs) + _gumbel_const()

    b = lax.bitcast_convert_type(scores, jnp.int32)
    kp = jnp.where(b >= 0, ~b & 0x7FFFFFFF, b).astype(jnp.int32)
    keys_pad = jnp.concatenate(
        [kp, jnp.full((NPAD - N,), -1, jnp.int32)])

    feat_copy = _tc_copy(features)
    mask_nodes, keep_nodes = _sc_sort(keys_pad)
    tok128 = jnp.broadcast_to(mask_token, (128, D))
    new_features = _sc_scatter(mask_nodes, feat_copy, tok128)
    return (new_features, mask_nodes, keep_nodes)


# speedup vs baseline: 1.0611x; 1.0172x over previous
import functools

import jax
import jax.numpy as jnp
from jax import lax
from jax.experimental import pallas as pl
from jax.experimental.pallas import tpu as pltpu
from jax.experimental.pallas import tpu_sc as plsc

MASK_RATE = 0.5
VIOLATION_WEIGHT = 0.3
RANDOM_WEIGHT = 0.7
N = 100000
D = 128
NUM_MASK = 50000

NT = 16
NL = 16
NV = 391
CH = NL * NV
NPAD = NT * CH
NTRASH = NPAD + NL
NCH = 50
KEEP_DUMMY = 50048
KEEP_BUF = KEEP_DUMMY + NT * NL

OUT_CHUNK = 3128
LAST_CHUNK = NUM_MASK - 15 * OUT_CHUNK


def _sc_body(keys_hbm, mask_hbm, keep_hbm,
             v_keys, v_vals, v_hist, v_offs, v_grid, v_t256, v_pre, v_base,
             v_pos, v_kbuf, v_vbuf, v_fbuf,
             sp_keys_a, sp_vals_a, sp_keys_b, sp_vals_b, sp_T, sp_ind,
             sp_keep, sp_cnt, sem_a, sem_b, sem_c, sem_d):
    _SEMS = (sem_a, sem_b, sem_c, sem_d)
    tid = lax.axis_index("s")
    iota = lax.iota(jnp.int32, NL)
    lane_sub = iota * NV
    lane_hist = iota * 256
    zeros16 = jnp.zeros((NL,), jnp.int32)
    ones16 = jnp.ones((NL,), jnp.int32)

    def _z(i, _):
        v_vals[pl.ds(i * NL, NL)] = zeros16
        return 0
    lax.fori_loop(0, NV, _z, 0, unroll=8)
    pltpu.sync_copy(v_vals.at[pl.ds(0, CH)], sp_ind.at[pl.ds(tid * CH, CH)])

    sp_bufs = [(sp_keys_a, sp_vals_a), (sp_keys_b, sp_vals_b)]

    for p in range(4):
        shift = 8 * p
        if p == 0:
            keys_src = None
        else:
            keys_src, vals_src = sp_bufs[(p + 1) % 2]
        keys_dst, vals_dst = sp_bufs[p % 2]

        if p == 0:
            cp_k = pltpu.make_async_copy(
                keys_hbm.at[pl.ds(tid * CH, CH)], v_keys, sem_c)
            cp_k.start()
            cp_v = None
        else:
            cp_k = pltpu.make_async_copy(
                keys_src.at[pl.ds(tid * CH, CH)], v_keys, sem_c)
            cp_k.start()
            cp_v = pltpu.make_async_copy(
                vals_src.at[pl.ds(tid * CH, CH)], v_vals, sem_d)
            cp_v.start()

        def _zh(i, _):
            v_hist[pl.ds(i * NL, NL)] = zeros16
            return 0
        lax.fori_loop(0, 256, _zh, 0, unroll=8)
        cp_k.wait()

        def _hist(v, _):
            k = plsc.load_gather(v_keys, [lane_sub + v])
            ku = plsc.bitcast(k, jnp.uint32)
            d = plsc.bitcast((ku >> shift) & 0xFF, jnp.int32)
            plsc.addupdate_scatter(v_hist, [lane_hist + d], ones16)
            return 0
        lax.fori_loop(0, NV, _hist, 0, unroll=8)
        if cp_v is not None:
            cp_v.wait()

        for dv in range(16):
            acc = v_hist[pl.ds(dv * NL, NL)]
            for l in range(1, NL):
                acc = acc + v_hist[pl.ds(l * 256 + dv * NL, NL)]
            v_t256[pl.ds(dv * NL, NL)] = acc
        pltpu.sync_copy(v_t256.at[pl.ds(0, 256)], sp_T.at[pl.ds(tid * 256, 256)])
        plsc.subcore_barrier()

        pltpu.sync_copy(sp_T.at[pl.ds(0, 4096)], v_grid)
        for dv in range(16):
            tot = zeros16
            pre = zeros16
            for t in range(NT):
                x = v_grid[pl.ds(t * 256 + dv * NL, NL)]
                tot = tot + x
                pre = pre + x * jnp.where(t < tid, 1, 0).astype(jnp.int32)
            v_t256[pl.ds(dv * NL, NL)] = tot
            v_pre[pl.ds(dv * NL, NL)] = pre
        carry = jnp.int32(0)
        for dv in range(16):
            g = v_t256[pl.ds(dv * NL, NL)]
            excl = plsc.cumsum(g) - g
            v_base[pl.ds(dv * NL, NL)] = excl + carry
            carry = carry + jnp.sum(g)
        for dv in range(16):
            run = v_base[pl.ds(dv * NL, NL)] + v_pre[pl.ds(dv * NL, NL)]
            for l in range(NL):
                v_offs[pl.ds(l * 256 + dv * NL, NL)] = run
                run = run + v_hist[pl.ds(l * 256 + dv * NL, NL)]

        last = p == 3
        pad_idx = NPAD + iota

        def _issue(par, sem):
            if not last:
                pltpu.async_copy(v_kbuf.at[par], keys_dst.at[v_pos.at[par]], sem)
            pltpu.async_copy(v_vbuf.at[par], vals_dst.at[v_pos.at[par]], sem)
            if last:
                pltpu.async_copy(v_fbuf.at[par], sp_ind.at[v_vbuf.at[par]], sem)

        def _drain(par, sem):
            if not last:
                pltpu.make_async_copy(v_kbuf.at[par], keys_dst.at[v_pos.at[par]], sem).wait()
            pltpu.make_async_copy(v_vbuf.at[par], vals_dst.at[v_pos.at[par]], sem).wait()
            if last:
                pltpu.make_async_copy(v_fbuf.at[par], sp_ind.at[v_vbuf.at[par]], sem).wait()

        def _chunk(c2, _):
            for par in range(2):
                sem = _SEMS[par]

                @pl.when(c2 > 0)
                def _w():
                    _drain(par, sem)

                for j in range(8):
                    v = (c2 * 2 + par) * 8 + j
                    valid = v < NV
                    vv = jnp.where(valid, v, 0)
                    sl = lane_sub + vv
                    k = plsc.load_gather(v_keys, [sl])
                    if p == 0:
                        val = tid * CH + sl
                    else:
                        val = plsc.load_gather(v_vals, [sl])
                    ku = plsc.bitcast(k, jnp.uint32)
                    d = plsc.bitcast((ku >> shift) & 0xFF, jnp.int32)
                    addr = lane_hist + d
                    pos = plsc.load_gather(v_offs, [addr])
                    vmul = jnp.where(valid, 1, 0).astype(jnp.int32)
                    plsc.store_scatter(v_offs, [addr], pos + vmul)
                    pos = jnp.where(valid, pos, pad_idx)
                    val = jnp.where(valid, val, pad_idx)
                    v_pos[par, pl.ds(j * NL, NL)] = pos
                    v_kbuf[par, pl.ds(j * NL, NL)] = k
                    v_vbuf[par, pl.ds(j * NL, NL)] = val
                    if last:
                        v_fbuf[par, pl.ds(j * NL, NL)] = jnp.where(
                            pos < NUM_MASK, 1, 0).astype(jnp.int32) * vmul
                _issue(par, sem)
            return 0
        lax.fori_loop(0, NCH // 2, _chunk, 0)
        for par in range(2):
            _drain(par, _SEMS[par])
        plsc.subcore_barrier()

    pltpu.sync_copy(sp_ind.at[pl.ds(tid * CH, CH)], v_keys)
    gbase = tid * CH

    def _cnt(i, c):
        ind = v_keys[pl.ds(i * NL, NL)]
        g = gbase + i * NL + iota
        f = jnp.where((ind == 0) & (g < N), 1, 0).astype(jnp.int32)
        return c + jnp.sum(f)
    cnt = lax.fori_loop(0, NV, _cnt, jnp.int32(0), unroll=4)
    v_t256[pl.ds(0, NL)] = jnp.where(iota == 0, cnt, 0).astype(jnp.int32)
    pltpu.sync_copy(v_t256.at[pl.ds(0, NL)], sp_cnt.at[pl.ds(tid * NL, NL)])
    plsc.subcore_barrier()
    pltpu.sync_copy(sp_cnt.at[pl.ds(0, 256)], v_t256.at[pl.ds(0, 256)])
    off = jnp.int32(0)
    for t in range(NT):
        x = v_t256[pl.ds(t * NL, NL)]
        off = off + jnp.sum(x) * jnp.where(t < tid, 1, 0).astype(jnp.int32)

    def _keep(c2, run):
        for par in range(2):
            sem = sem_a if par == 0 else sem_b

            @pl.when(c2 > 0)
            def _wk():
                pltpu.make_async_copy(
                    v_vbuf.at[par], sp_keep.at[v_pos.at[par]], sem).wait()

            for j in range(8):
                i = (c2 * 2 + par) * 8 + j
                valid = i < NV
                ii = jnp.where(valid, i, 0)
                ind = plsc.load_gather(v_keys, [ii * NL + iota])
                g = gbase + ii * NL + iota
                f = jnp.where((ind == 0) & (g < N) & valid, 1, 0).astype(jnp.int32)
                excl = plsc.cumsum(f) - f
                pos = run + excl
                pos = jnp.where(f == 1, pos, KEEP_DUMMY + tid * NL + iota)
                run = run + jnp.sum(f)
                v_pos[par, pl.ds(j * NL, NL)] = pos
                v_vbuf[par, pl.ds(j * NL, NL)] = g
            pltpu.async_copy(v_vbuf.at[par], sp_keep.at[v_pos.at[par]], sem)
        return run
    lax.fori_loop(0, NCH // 2, _keep, off)
    pltpu.make_async_copy(v_vbuf.at[0], sp_keep.at[v_pos.at[0]], sem_a).wait()
    pltpu.make_async_copy(v_vbuf.at[1], sp_keep.at[v_pos.at[1]], sem_b).wait()
    plsc.subcore_barrier()

    _, sp_vals_fin = sp_bufs[1]

    @pl.when(tid < 15)
    def _copy_main():
        o = pl.multiple_of(tid * OUT_CHUNK, 8)
        pltpu.sync_copy(sp_vals_fin.at[pl.ds(o, OUT_CHUNK)], v_vals.at[pl.ds(0, OUT_CHUNK)])
        pltpu.sync_copy(v_vals.at[pl.ds(0, OUT_CHUNK)], mask_hbm.at[pl.ds(o, OUT_CHUNK)])
        pltpu.sync_copy(sp_keep.at[pl.ds(o, OUT_CHUNK)], v_keys.at[pl.ds(0, OUT_CHUNK)])
        pltpu.sync_copy(v_keys.at[pl.ds(0, OUT_CHUNK)], keep_hbm.at[pl.ds(o, OUT_CHUNK)])

    @pl.when(tid == 15)
    def _copy_last():
        o = 15 * OUT_CHUNK
        pltpu.sync_copy(sp_vals_fin.at[pl.ds(o, LAST_CHUNK)], v_vals.at[pl.ds(0, LAST_CHUNK)])
        pltpu.sync_copy(v_vals.at[pl.ds(0, LAST_CHUNK)], mask_hbm.at[pl.ds(o, LAST_CHUNK)])
        pltpu.sync_copy(sp_keep.at[pl.ds(o, LAST_CHUNK)], v_keys.at[pl.ds(0, LAST_CHUNK)])
        pltpu.sync_copy(v_keys.at[pl.ds(0, LAST_CHUNK)], keep_hbm.at[pl.ds(o, LAST_CHUNK)])


@functools.partial(jax.jit, static_argnums=())
def _sc_sort(keys_pad):
    mesh = plsc.VectorSubcoreMesh(core_axis_name="c", subcore_axis_name="s",
                                  num_cores=1)
    f = pl.kernel(
        _sc_body,
        compiler_params=pltpu.CompilerParams(needs_layout_passes=False),
        cost_estimate=pl.CostEstimate(
            flops=4_000_000, bytes_accessed=16_000_000, transcendentals=0),
        out_type=(
            jax.ShapeDtypeStruct((NUM_MASK,), jnp.int32),
            jax.ShapeDtypeStruct((NUM_MASK,), jnp.int32),
        ),
        mesh=mesh,
        scratch_types=dict(
            v_keys=pltpu.VMEM((CH,), jnp.int32),
            v_vals=pltpu.VMEM((CH,), jnp.int32),
            v_hist=pltpu.VMEM((4096,), jnp.int32),
            v_offs=pltpu.VMEM((4096,), jnp.int32),
            v_grid=pltpu.VMEM((4096,), jnp.int32),
            v_t256=pltpu.VMEM((256,), jnp.int32),
            v_pre=pltpu.VMEM((256,), jnp.int32),
            v_base=pltpu.VMEM((256,), jnp.int32),
            v_pos=pltpu.VMEM((4, 128), jnp.int32),
            v_kbuf=pltpu.VMEM((4, 128), jnp.int32),
            v_vbuf=pltpu.VMEM((4, 128), jnp.int32),
            v_fbuf=pltpu.VMEM((4, 128), jnp.int32),
            sp_keys_a=pltpu.VMEM_SHARED((NTRASH,), jnp.int32),
            sp_vals_a=pltpu.VMEM_SHARED((NTRASH,), jnp.int32),
            sp_keys_b=pltpu.VMEM_SHARED((NTRASH,), jnp.int32),
            sp_vals_b=pltpu.VMEM_SHARED((NTRASH,), jnp.int32),
            sp_T=pltpu.VMEM_SHARED((4096,), jnp.int32),
            sp_ind=pltpu.VMEM_SHARED((NTRASH,), jnp.int32),
            sp_keep=pltpu.VMEM_SHARED((KEEP_BUF,), jnp.int32),
            sp_cnt=pltpu.VMEM_SHARED((256,), jnp.int32),
            sem_a=pltpu.SemaphoreType.DMA,
            sem_b=pltpu.SemaphoreType.DMA,
            sem_c=pltpu.SemaphoreType.DMA,
            sem_d=pltpu.SemaphoreType.DMA,
        ),
    )
    return f(keys_pad)


def _tc_copy_body(f_ref, o_ref):
    o_ref[...] = f_ref[...]


def _tc_copy(features):
    blk = 2000
    grid = (N // blk,)
    return pl.pallas_call(
        _tc_copy_body,
        grid=grid,
        in_specs=[pl.BlockSpec((blk, D), lambda i: (i, 0))],
        out_specs=pl.BlockSpec((blk, D), lambda i: (i, 0)),
        out_shape=jax.ShapeDtypeStruct((N, D), jnp.float32),
    )(features)


NW = 32
NFULL_CHUNK = NUM_MASK // 128
REM = NUM_MASK - NFULL_CHUNK * 128
REM_W = NFULL_CHUNK % NW
SCAT_IT = (NFULL_CHUNK + NW - 1) // NW


def _scatter_body(mask_hbm, feat_in_hbm, tok_hbm, out_hbm, v_idx, v_rem,
                  v_tok, sem, sem_r):
    del feat_in_hbm
    wid = lax.axis_index("s") * 2 + lax.axis_index("c")
    pltpu.sync_copy(tok_hbm, v_tok)

    def _go(i, _):
        c = wid + i * NW

        @pl.when(c < NFULL_CHUNK)
        def _full():
            pltpu.sync_copy(mask_hbm.at[pl.ds(c * 128, 128)], v_idx.at[i])
            pltpu.async_copy(v_tok, out_hbm.at[v_idx.at[i]], sem)
        return 0
    lax.fori_loop(0, SCAT_IT, _go, 0)

    @pl.when(wid == REM_W)
    def _rem():
        o = pl.multiple_of(NFULL_CHUNK * 128, 8)
        pltpu.sync_copy(mask_hbm.at[pl.ds(o, REM)], v_rem)
        pltpu.async_copy(v_tok.at[pl.ds(0, REM)], out_hbm.at[v_rem], sem_r).wait()

    def _drain(i, _):
        c = wid + i * NW

        @pl.when(c < NFULL_CHUNK)
        def _w():
            pltpu.make_async_copy(v_tok, out_hbm.at[v_idx.at[i]], sem).wait()
        return 0
    lax.fori_loop(0, SCAT_IT, _drain, 0)


def _sc_scatter(mask_nodes, feat_copy, tok128):
    from jax._src.pallas import mpmd as _mpmd
    mesh = plsc.VectorSubcoreMesh(core_axis_name="c", subcore_axis_name="s",
                                  num_cores=2)
    f = _mpmd._mpmd_map(
        [(mesh, _scatter_body)],
        (jax.ShapeDtypeStruct((N, D), jnp.float32),),
        input_output_aliases={1: 0},
        compiler_params=pltpu.CompilerParams(needs_layout_passes=False),
        scratch_types=dict(
            v_idx=pltpu.VMEM((SCAT_IT, 128), jnp.int32),
            v_rem=pltpu.VMEM((REM,), jnp.int32),
            v_tok=pltpu.VMEM((128, D), jnp.float32),
            sem=pltpu.SemaphoreType.DMA,
            sem_r=pltpu.SemaphoreType.DMA,
        ),
    )
    (out,) = f(mask_nodes, feat_copy, tok128)
    return out


@functools.lru_cache(maxsize=1)
def _gumbel_const():
    skey = jax.random.key(42)
    return jax.random.gumbel(skey, (N,), dtype=jnp.float32)


def kernel(features, cic_scores, mask_token):
    weights = jnp.array([0.25, 0.25, 0.25, 0.25], dtype=jnp.float32)
    weighted = 1.0 - weights[None, :] * jnp.clip(cic_scores, 0.0, 1.0)
    total_scores = 1.0 - jnp.prod(weighted, axis=1)
    total_scores = jnp.clip(total_scores.astype(jnp.float32), 0.0, 1.0)
    violation_probs = total_scores + 1e-06
    random_probs = jnp.ones(N, dtype=jnp.float32)
    probs = VIOLATION_WEIGHT * violation_probs + RANDOM_WEIGHT * random_probs
    probs = probs / jnp.sum(probs)
    scores = jnp.log(probs) + _gumbel_const()

    b = lax.bitcast_convert_type(scores, jnp.int32)
    kp = jnp.where(b >= 0, ~b & 0x7FFFFFFF, b).astype(jnp.int32)
    keys_pad = jnp.concatenate(
        [kp, jnp.full((NPAD - N,), -1, jnp.int32)])

    feat_copy = _tc_copy(features)
    mask_nodes, keep_nodes = _sc_sort(keys_pad)
    tok128 = jnp.broadcast_to(mask_token, (128, D))
    new_features = _sc_scatter(mask_nodes, feat_copy, tok128)
    return (new_features, mask_nodes, keep_nodes)
